# baseline (speedup 1.0000x reference)
import functools

import jax
import jax.numpy as jnp
from jax import lax
from jax.experimental import pallas as pl
from jax.experimental.pallas import tpu as pltpu
from jax.experimental.pallas import tpu_sc as plsc

L = 16
NW = 32


def _sc_partials(C, HW, K, table, indf, maskf, tgtt):
    P = (indf.shape[0]) // NW
    G = P // L
    NIDX = P * C
    RPG = (L * C) // 128
    BPW = P // K
    KB = K.bit_length() - 1

    mesh = plsc.VectorSubcoreMesh(core_axis_name="c", subcore_axis_name="s")

    @functools.partial(
        pl.kernel,
        compiler_params=pltpu.CompilerParams(needs_layout_passes=False),
        out_type=(
            jax.ShapeDtypeStruct((NW, L), jnp.float32),
            jax.ShapeDtypeStruct((NW, L), jnp.float32),
        ),
        mesh=mesh,
        scratch_types=[
            pltpu.VMEM((P,), jnp.int32),
            pltpu.VMEM((P,), jnp.float32),
            pltpu.VMEM((P * C,), jnp.float32),
            pltpu.VMEM((G * RPG, 128), jnp.int32),
            pltpu.VMEM((NIDX,), jnp.float32),
            pltpu.VMEM((L,), jnp.float32),
            pltpu.VMEM((L,), jnp.float32),
            pltpu.SemaphoreType.DMA,
            [pltpu.SemaphoreType.DMA] * G,
        ],
    )
    def k(table_h, ind_h, mask_h, tgt_h, oacc_h, omacc_h,
          ind_v, mask_v, tgt_v, idx_v, pbuf, acc_v, macc_v, sem_in, gsems):
        nc = 2
        wid = lax.axis_index("s") * nc + lax.axis_index("c")
        base = wid * P
        pltpu.sync_copy(ind_h.at[pl.ds(base, P)], ind_v)
        cp_t = pltpu.async_copy(
            tgt_h.at[pl.ds(base * C, P * C)], tgt_v, sem_in)
        cp_m = pltpu.async_copy(mask_h.at[pl.ds(base, P)], mask_v, sem_in)

        iota = lax.iota(jnp.int32, L)

        cps = []
        for g in range(G):
            ind_vec = ind_v[pl.ds(g * L, L)]
            pair_vec = base + g * L + iota
            rowb = (pair_vec >> KB) * (C * HW) + ind_vec
            for c in range(C):
                q = g * (L * C) + c * L
                idx_v[q // 128, pl.ds(q % 128, L)] = rowb + c * HW
            pair = []
            for r in range(RPG):
                row = g * RPG + r
                pair.append(pltpu.async_copy(
                    table_h.at[idx_v.at[row]],
                    pbuf.at[pl.ds(row * 128, 128)], gsems[g]))
            cps.append(pair)

        cp_t.wait()
        cp_m.wait()

        acc = jnp.zeros((L,), jnp.float32)
        macc = jnp.zeros((L,), jnp.float32)
        for g in range(G):
            for cp in cps[g]:
                cp.wait()
            mask_vec = mask_v[pl.ds(g * L, L)]
            for c in range(C):
                pred = pbuf[pl.ds(g * (L * C) + c * L, L)]
                tgt = plsc.load_gather(tgt_v, [g * (L * C) + iota * C + c])
                d = jnp.abs(pred - tgt)
                elem = jnp.where(d < 1.0, 0.5 * d * d, d - 0.5)
                acc = acc + elem * mask_vec
            macc = macc + mask_vec

        acc_v[...] = acc
        macc_v[...] = macc
        pltpu.sync_copy(acc_v, oacc_h.at[wid])
        pltpu.sync_copy(macc_v, omacc_h.at[wid])

    return k(table, indf, maskf, tgtt)


def _combine(C, acc_ref, macc_ref, o_ref):
    s = jnp.sum(acc_ref[...])
    m = jnp.sum(macc_ref[...]) * C
    o_ref[...] = jnp.broadcast_to(s / m, (1, 1))


def kernel(output, mask, ind, target):
    B, C, H, W = output.shape
    K = ind.shape[1]

    table = output.reshape(-1)
    indf = ind.reshape(-1)
    maskf = mask.reshape(-1)
    tgtt = target.reshape(-1)

    oacc, omacc = _sc_partials(C, H * W, K, table, indf, maskf, tgtt)
    out = pl.pallas_call(
        functools.partial(_combine, float(C)),
        out_shape=jax.ShapeDtypeStruct((1, 1), jnp.float32),
    )(oacc, omacc)
    return out[0, 0]


# speedup vs baseline: 1.0105x; 1.0105x over previous
import functools

import jax
import jax.numpy as jnp
from jax import lax
from jax.experimental import pallas as pl
from jax.experimental.pallas import tpu as pltpu
from jax.experimental.pallas import tpu_sc as plsc

L = 16
NW = 32


def _sc_partials(C, HW, K, table, indf, maskf, tgtt):
    P = (indf.shape[0]) // NW
    G = P // L
    NIDX = P * C
    RPG = (L * C) // 128
    BPW = P // K
    KB = K.bit_length() - 1

    mesh = plsc.VectorSubcoreMesh(core_axis_name="c", subcore_axis_name="s")

    @functools.partial(
        pl.kernel,
        compiler_params=pltpu.CompilerParams(needs_layout_passes=False),
        out_type=jax.ShapeDtypeStruct((NW, 2 * L), jnp.float32),
        mesh=mesh,
        scratch_types=[
            pltpu.VMEM((P,), jnp.int32),
            pltpu.VMEM((P,), jnp.float32),
            pltpu.VMEM((P * C,), jnp.float32),
            pltpu.VMEM((G * RPG, 128), jnp.int32),
            pltpu.VMEM((NIDX,), jnp.float32),
            pltpu.VMEM((2 * L,), jnp.float32),
            pltpu.SemaphoreType.DMA,
            pltpu.SemaphoreType.DMA,
            pltpu.SemaphoreType.DMA,
            [pltpu.SemaphoreType.DMA] * G,
        ],
    )
    def k(table_h, ind_h, mask_h, tgt_h, out_h,
          ind_v, mask_v, tgt_v, idx_v, pbuf, acc_v, sem_in, sem_h, sem_t,
          gsems):
        nc = 2
        wid = lax.axis_index("s") * nc + lax.axis_index("c")
        base = wid * P
        HG = 4
        cp_h = pltpu.async_copy(
            ind_h.at[pl.ds(base, HG * L)], ind_v.at[pl.ds(0, HG * L)], sem_h)
        cp_r = pltpu.async_copy(
            ind_h.at[pl.ds(base + HG * L, P - HG * L)],
            ind_v.at[pl.ds(HG * L, P - HG * L)], sem_t)
        cp_t = pltpu.async_copy(
            tgt_h.at[pl.ds(base * C, P * C)], tgt_v, sem_in)
        cp_m = pltpu.async_copy(mask_h.at[pl.ds(base, P)], mask_v, sem_in)
        cp_h.wait()

        iota = lax.iota(jnp.int32, L)

        cps = []
        for g in range(G):
            if g == HG:
                cp_r.wait()
            ind_vec = ind_v[pl.ds(g * L, L)]
            pair_vec = base + g * L + iota
            rowb = (pair_vec >> KB) * (C * HW) + ind_vec
            for c in range(C):
                q = g * (L * C) + c * L
                idx_v[q // 128, pl.ds(q % 128, L)] = rowb + c * HW
            pair = []
            for r in range(RPG):
                row = g * RPG + r
                pair.append(pltpu.async_copy(
                    table_h.at[idx_v.at[row]],
                    pbuf.at[pl.ds(row * 128, 128)], gsems[g]))
            cps.append(pair)

        cp_t.wait()
        cp_m.wait()

        acc = jnp.zeros((L,), jnp.float32)
        macc = jnp.zeros((L,), jnp.float32)
        for g in range(G):
            for cp in cps[g]:
                cp.wait()
            mask_vec = mask_v[pl.ds(g * L, L)]
            for c in range(C):
                pred = pbuf[pl.ds(g * (L * C) + c * L, L)]
                tgt = plsc.load_gather(tgt_v, [g * (L * C) + iota * C + c])
                d = jnp.abs(pred - tgt)
                elem = jnp.where(d < 1.0, 0.5 * d * d, d - 0.5)
                acc = acc + elem * mask_vec
            macc = macc + mask_vec

        acc_v[pl.ds(0, L)] = acc
        acc_v[pl.ds(L, L)] = macc
        pltpu.sync_copy(acc_v, out_h.at[wid])

    return k(table, indf, maskf, tgtt)


def _combine(C, part_ref, o_ref):
    s = jnp.sum(part_ref[:, :16])
    m = jnp.sum(part_ref[:, 16:]) * C
    o_ref[...] = jnp.broadcast_to(s / m, (1, 1))


def kernel(output, mask, ind, target):
    B, C, H, W = output.shape
    K = ind.shape[1]

    table = output.reshape(-1)
    indf = ind.reshape(-1)
    maskf = mask.reshape(-1)
    tgtt = target.reshape(-1)

    parts = _sc_partials(C, H * W, K, table, indf, maskf, tgtt)
    out = pl.pallas_call(
        functools.partial(_combine, float(C)),
        out_shape=jax.ShapeDtypeStruct((1, 1), jnp.float32),
    )(parts)
    return out[0, 0]


# speedup vs baseline: 1.0298x; 1.0191x over previous
import functools

import jax
import jax.numpy as jnp
from jax import lax
from jax.experimental import pallas as pl
from jax.experimental.pallas import tpu as pltpu
from jax.experimental.pallas import tpu_sc as plsc

L = 16
NW = 32


def _sc_partials(C, HW, K, table, indf, maskf, tgtt):
    P = (indf.shape[0]) // NW
    G = P // L
    NIDX = P * C
    RPG = (L * C) // 128
    BPW = P // K
    KB = K.bit_length() - 1

    mesh = plsc.VectorSubcoreMesh(core_axis_name="c", subcore_axis_name="s")

    @functools.partial(
        pl.kernel,
        compiler_params=pltpu.CompilerParams(needs_layout_passes=False),
        out_type=jax.ShapeDtypeStruct((NW, 2 * L), jnp.float32),
        mesh=mesh,
        scratch_types=[
            pltpu.VMEM((P,), jnp.int32),
            pltpu.VMEM((P,), jnp.float32),
            pltpu.VMEM((P * C,), jnp.float32),
            pltpu.VMEM((G * RPG, 128), jnp.int32),
            pltpu.VMEM((NIDX,), jnp.float32),
            pltpu.VMEM((2 * L,), jnp.float32),
            pltpu.SemaphoreType.DMA,
            pltpu.SemaphoreType.DMA,
            pltpu.SemaphoreType.DMA,
            [pltpu.SemaphoreType.DMA] * G,
        ],
    )
    def k(table_h, ind_h, mask_h, tgt_h, out_h,
          ind_v, mask_v, tgt_v, idx_v, pbuf, acc_v, sem_in, sem_h, sem_t,
          gsems):
        nc = 2
        wid = lax.axis_index("s") * nc + lax.axis_index("c")
        base = wid * P
        HG = 4
        cp_h = pltpu.async_copy(
            ind_h.at[pl.ds(base, HG * L)], ind_v.at[pl.ds(0, HG * L)], sem_h)
        cp_r = pltpu.async_copy(
            ind_h.at[pl.ds(base + HG * L, P - HG * L)],
            ind_v.at[pl.ds(HG * L, P - HG * L)], sem_t)
        cp_t = pltpu.async_copy(
            tgt_h.at[pl.ds(base * C, P * C)], tgt_v, sem_in)
        cp_m = pltpu.async_copy(mask_h.at[pl.ds(base, P)], mask_v, sem_in)
        cp_h.wait()

        iota = lax.iota(jnp.int32, L)

        cps = []
        for g in range(G):
            if g == HG:
                cp_r.wait()
            ind_vec = ind_v[pl.ds(g * L, L)]
            pair_vec = base + g * L + iota
            rowb = (pair_vec >> KB) * (C * HW) + ind_vec
            for c in range(C):
                q = g * (L * C) + c * L
                idx_v[q // 128, pl.ds(q % 128, L)] = rowb + c * HW
            pair = []
            for r in range(RPG):
                row = g * RPG + r
                pair.append(pltpu.async_copy(
                    table_h.at[idx_v.at[row]],
                    pbuf.at[pl.ds(row * 128, 128)], gsems[g]))
            cps.append(pair)

        cp_t.wait()
        cp_m.wait()

        acc = jnp.zeros((L,), jnp.float32)
        macc = jnp.zeros((L,), jnp.float32)
        for g in range(G):
            for cp in cps[g]:
                cp.wait()
            mask_vec = mask_v[pl.ds(g * L, L)]
            gpb = g // (G // BPW)
            gk = g % (G // BPW)
            for c in range(C):
                pred = pbuf[pl.ds(g * (L * C) + c * L, L)]
                tgt = tgt_v[pl.ds(gpb * (C * K) + c * K + gk * L, L)]
                d = jnp.abs(pred - tgt)
                elem = jnp.where(d < 1.0, 0.5 * d * d, d - 0.5)
                acc = acc + elem * mask_vec
            macc = macc + mask_vec

        acc_v[pl.ds(0, L)] = acc
        acc_v[pl.ds(L, L)] = macc
        pltpu.sync_copy(acc_v, out_h.at[wid])

    return k(table, indf, maskf, tgtt)


def _combine(C, part_ref, o_ref):
    s = jnp.sum(part_ref[:, :16])
    m = jnp.sum(part_ref[:, 16:]) * C
    o_ref[...] = jnp.broadcast_to(s / m, (1, 1))


def kernel(output, mask, ind, target):
    B, C, H, W = output.shape
    K = ind.shape[1]

    table = output.reshape(-1)
    indf = ind.reshape(-1)
    maskf = mask.reshape(-1)
    tgtt = jnp.transpose(target, (0, 2, 1)).reshape(-1)

    parts = _sc_partials(C, H * W, K, table, indf, maskf, tgtt)
    out = pl.pallas_call(
        functools.partial(_combine, float(C)),
        out_shape=jax.ShapeDtypeStruct((1, 1), jnp.float32),
    )(parts)
    return out[0, 0]


# speedup vs baseline: 1.0588x; 1.0282x over previous
import functools

import jax
import jax.numpy as jnp
from jax import lax
from jax.experimental import pallas as pl
from jax.experimental.pallas import tpu as pltpu
from jax.experimental.pallas import tpu_sc as plsc

L = 16
NW = 32
NCH = 4


def _sc_partials(C, HW, K, table, indf, maskf, tgtt):
    P = (indf.shape[0]) // NW
    G = P // L
    NIDX = P * C
    RPG = (L * C) // 128
    BPW = P // K
    KB = K.bit_length() - 1
    GPC = G // NCH

    mesh = plsc.VectorSubcoreMesh(core_axis_name="c", subcore_axis_name="s")

    @functools.partial(
        pl.kernel,
        compiler_params=pltpu.CompilerParams(needs_layout_passes=False),
        out_type=jax.ShapeDtypeStruct((NW, 2 * L), jnp.float32),
        mesh=mesh,
        scratch_types=[
            pltpu.VMEM((P,), jnp.int32),
            pltpu.VMEM((P,), jnp.float32),
            pltpu.VMEM((P * C,), jnp.float32),
            pltpu.VMEM((G * RPG, 128), jnp.int32),
            pltpu.VMEM((NIDX,), jnp.float32),
            pltpu.VMEM((2 * L,), jnp.float32),
            pltpu.SemaphoreType.DMA,
            pltpu.SemaphoreType.DMA,
            pltpu.SemaphoreType.DMA,
            [pltpu.SemaphoreType.DMA] * NCH,
        ],
    )
    def k(table_h, ind_h, mask_h, tgt_h, out_h,
          ind_v, mask_v, tgt_v, idx_v, pbuf, acc_v, sem_in, sem_h, sem_t,
          gsems):
        nc = 2
        wid = lax.axis_index("s") * nc + lax.axis_index("c")
        base = wid * P
        HG = GPC
        cp_h = pltpu.async_copy(
            ind_h.at[pl.ds(base, HG * L)], ind_v.at[pl.ds(0, HG * L)], sem_h)
        cp_r = pltpu.async_copy(
            ind_h.at[pl.ds(base + HG * L, P - HG * L)],
            ind_v.at[pl.ds(HG * L, P - HG * L)], sem_t)
        cp_t = pltpu.async_copy(
            tgt_h.at[pl.ds(base * C, P * C)], tgt_v, sem_in)
        cp_m = pltpu.async_copy(mask_h.at[pl.ds(base, P)], mask_v, sem_in)
        cp_h.wait()

        iota = lax.iota(jnp.int32, L)

        for ch in range(NCH):
            if ch == 1:
                cp_r.wait()

            def bbody(g, _, _sem=gsems[ch]):
                ind_vec = ind_v[pl.ds(g * L, L)]
                pair_vec = base + g * L + iota
                rowb = (pair_vec >> KB) * (C * HW) + ind_vec
                for c in range(C):
                    row = g * RPG + (c * L) // 128
                    idx_v[row, pl.ds((c * L) % 128, L)] = rowb + c * HW
                for r in range(RPG):
                    row = g * RPG + r
                    pltpu.async_copy(
                        table_h.at[idx_v.at[row]],
                        pbuf.at[pl.ds(row * 128, 128)], _sem)
                return 0

            lax.fori_loop(ch * GPC, (ch + 1) * GPC, bbody, 0)

        cp_t.wait()
        cp_m.wait()

        acc = jnp.zeros((L,), jnp.float32)
        macc = jnp.zeros((L,), jnp.float32)
        for ch in range(NCH):
            for g in range(ch * GPC, (ch + 1) * GPC):
                for r in range(RPG):
                    row = g * RPG + r
                    pltpu.make_async_copy(
                        table_h.at[idx_v.at[row]],
                        pbuf.at[pl.ds(row * 128, 128)], gsems[ch]).wait()

            def cbody(g, carry):
                acc, macc = carry
                mask_vec = mask_v[pl.ds(g * L, L)]
                gpb = g // (G // BPW)
                gk = g % (G // BPW)
                for c in range(C):
                    pred = pbuf[pl.ds(g * (L * C) + c * L, L)]
                    tgt = tgt_v[pl.ds(gpb * (C * K) + c * K + gk * L, L)]
                    d = jnp.abs(pred - tgt)
                    elem = jnp.where(d < 1.0, 0.5 * d * d, d - 0.5)
                    acc = acc + elem * mask_vec
                macc = macc + mask_vec
                return acc, macc

            acc, macc = lax.fori_loop(
                ch * GPC, (ch + 1) * GPC, cbody, (acc, macc))

        acc_v[pl.ds(0, L)] = acc
        acc_v[pl.ds(L, L)] = macc
        pltpu.sync_copy(acc_v, out_h.at[wid])

    return k(table, indf, maskf, tgtt)


def _combine(C, part_ref, o_ref):
    s = jnp.sum(part_ref[:, :16])
    m = jnp.sum(part_ref[:, 16:]) * C
    o_ref[...] = jnp.broadcast_to(s / m, (1, 1))


def kernel(output, mask, ind, target):
    B, C, H, W = output.shape
    K = ind.shape[1]
    table = output.reshape(-1)
    indf = ind.reshape(-1)
    maskf = mask.reshape(-1)
    tgtt = jnp.transpose(target, (0, 2, 1)).reshape(-1)
    parts = _sc_partials(C, H * W, K, table, indf, maskf, tgtt)
    out = pl.pallas_call(
        functools.partial(_combine, float(C)),
        out_shape=jax.ShapeDtypeStruct((1, 1), jnp.float32),
    )(parts)
    return out[0, 0]


# speedup vs baseline: 1.0655x; 1.0063x over previous
import functools

import jax
import jax.numpy as jnp
from jax import lax
from jax.experimental import pallas as pl
from jax.experimental.pallas import tpu as pltpu
from jax.experimental.pallas import tpu_sc as plsc

L = 16
NW = 32
NCH = 4


def _sc_partials(C, HW, K, table, indf, maskf, tgtt):
    P = (indf.shape[0]) // NW
    G = P // L
    NIDX = P * C
    RPG = (L * C) // 128
    BPW = P // K
    KB = K.bit_length() - 1
    GPC = G // NCH

    mesh = plsc.VectorSubcoreMesh(core_axis_name="c", subcore_axis_name="s")

    @functools.partial(
        pl.kernel,
        compiler_params=pltpu.CompilerParams(needs_layout_passes=False),
        out_type=jax.ShapeDtypeStruct((NW, 2 * L), jnp.float32),
        mesh=mesh,
        scratch_types=[
            pltpu.VMEM((P,), jnp.int32),
            pltpu.VMEM((P,), jnp.float32),
            pltpu.VMEM((P * C,), jnp.float32),
            pltpu.VMEM((G * RPG, 128), jnp.int32),
            pltpu.VMEM((NIDX,), jnp.float32),
            pltpu.VMEM((2 * L,), jnp.float32),
            pltpu.SemaphoreType.DMA,
            pltpu.SemaphoreType.DMA,
            pltpu.SemaphoreType.DMA,
            [pltpu.SemaphoreType.DMA] * NCH,
        ],
    )
    def k(table_h, ind_h, mask_h, tgt_h, out_h,
          ind_v, mask_v, tgt_v, idx_v, pbuf, acc_v, sem_in, sem_h, sem_t,
          gsems):
        nc = 2
        wid = lax.axis_index("s") * nc + lax.axis_index("c")
        base = wid * P
        HG = GPC
        cp_h = pltpu.async_copy(
            ind_h.at[pl.ds(base, HG * L)], ind_v.at[pl.ds(0, HG * L)], sem_h)
        cp_r = pltpu.async_copy(
            ind_h.at[pl.ds(base + HG * L, P - HG * L)],
            ind_v.at[pl.ds(HG * L, P - HG * L)], sem_t)
        cp_t = pltpu.async_copy(
            tgt_h.at[pl.ds(base * C, P * C)], tgt_v, sem_in)
        cp_m = pltpu.async_copy(mask_h.at[pl.ds(base, P)], mask_v, sem_in)
        cp_h.wait()

        iota = lax.iota(jnp.int32, L)

        for ch in range(NCH):
            if ch == 1:
                cp_r.wait()

            def bbody(g, _, _sem=gsems[ch]):
                ind_vec = ind_v[pl.ds(g * L, L)]
                pair_vec = base + g * L + iota
                rowb = (pair_vec >> KB) * (C * HW) + ind_vec
                for c in range(C):
                    row = g * RPG + (c * L) // 128
                    idx_v[row, pl.ds((c * L) % 128, L)] = rowb + c * HW
                for r in range(RPG):
                    row = g * RPG + r
                    pltpu.async_copy(
                        table_h.at[idx_v.at[row]],
                        pbuf.at[pl.ds(row * 128, 128)], _sem)
                return 0

            lax.fori_loop(ch * GPC, (ch + 1) * GPC, bbody, 0)

        cp_t.wait()
        cp_m.wait()

        acc = jnp.zeros((L,), jnp.float32)
        macc = jnp.zeros((L,), jnp.float32)
        for ch in range(NCH):
            for g in range(ch * GPC, (ch + 1) * GPC):
                for r in range(RPG):
                    row = g * RPG + r
                    pltpu.make_async_copy(
                        table_h.at[idx_v.at[row]],
                        pbuf.at[pl.ds(row * 128, 128)], gsems[ch]).wait()

            def cbody(g, carry):
                acc0, macc0 = carry
                mask_vec = mask_v[pl.ds(g * L, L)]
                gpb = g // (G // BPW)
                gk = g % (G // BPW)
                pb = g * (L * C)
                tb = gpb * (C * K) + gk * L

                def ccbody(c, a):
                    pred = pbuf[pl.ds(pb + c * L, L)]
                    tgt = tgt_v[pl.ds(tb + c * K, L)]
                    d = jnp.abs(pred - tgt)
                    elem = jnp.where(d < 1.0, 0.5 * d * d, d - 0.5)
                    return a + elem * mask_vec

                acc0 = lax.fori_loop(0, C, ccbody, acc0)
                return acc0, macc0 + mask_vec

            acc, macc = lax.fori_loop(
                ch * GPC, (ch + 1) * GPC, cbody, (acc, macc))

        acc_v[pl.ds(0, L)] = acc
        acc_v[pl.ds(L, L)] = macc
        pltpu.sync_copy(acc_v, out_h.at[wid])

    return k(table, indf, maskf, tgtt)


def _combine(C, part_ref, o_ref):
    s = jnp.sum(part_ref[:, :16])
    m = jnp.sum(part_ref[:, 16:]) * C
    o_ref[...] = jnp.broadcast_to(s / m, (1, 1))


def kernel(output, mask, ind, target):
    B, C, H, W = output.shape
    K = ind.shape[1]
    table = output.reshape(-1)
    indf = ind.reshape(-1)
    maskf = mask.reshape(-1)
    tgtt = jnp.transpose(target, (0, 2, 1)).reshape(-1)
    parts = _sc_partials(C, H * W, K, table, indf, maskf, tgtt)
    out = pl.pallas_call(
        functools.partial(_combine, float(C)),
        out_shape=jax.ShapeDtypeStruct((1, 1), jnp.float32),
    )(parts)
    return out[0, 0]
